# Initial kernel scaffold; baseline (speedup 1.0000x reference)
import jax
import jax.numpy as jnp
from jax.experimental import pallas as pl


def kernel(x, edge_index, W1, b1, W2, b2):
    raise NotImplementedError("write your pallas kernel here")


# speedup vs baseline: 29.4405x; 29.4405x over previous
import functools

import jax
import jax.numpy as jnp
from jax import lax
from jax.experimental import pallas as pl
from jax.experimental.pallas import tpu as pltpu
from jax.experimental.pallas import tpu_sc as plsc

NC = 2
NS = 16
LANES = 16
CHUNK = 128


def _fill_rid(rid_v, nm, base):
    lane = lax.iota(jnp.int32, LANES)
    kpc = CHUNK // LANES

    def f(t, _):
        j = t // kpc
        k = t % kpc
        rid_v[j, pl.ds(k * LANES, LANES)] = lane + t * LANES + base
        return 0

    lax.fori_loop(0, nm * kpc, f, 0)


def _zero_rows(buf, nrows, width):
    zeros = jnp.zeros((LANES,), jnp.float32)
    kpr = width // LANES

    def f(t, _):
        i = t // kpr
        k = t % kpr
        buf[i, pl.ds(k * LANES, LANES)] = zeros
        return 0

    lax.fori_loop(0, nrows * kpr, f, 0)


def _make_deg(npad, c32):
    rpt = npad // NS
    nm = rpt // CHUNK
    mesh = plsc.VectorSubcoreMesh(core_axis_name="c", subcore_axis_name="s")

    @functools.partial(
        pl.kernel,
        mesh=mesh,
        out_type=jax.ShapeDtypeStruct((NC, npad, LANES), jnp.float32),
        compiler_params=pltpu.CompilerParams(use_tc_tiling_on_sc=False),
        scratch_types=[
            pltpu.VMEM((c32, CHUNK), jnp.int32),
            pltpu.VMEM((nm, CHUNK), jnp.int32),
            pltpu.VMEM((CHUNK, LANES), jnp.float32),
            pltpu.VMEM((CHUNK, LANES), jnp.float32),
            pltpu.VMEM_SHARED((npad, LANES), jnp.float32),
            pltpu.SemaphoreType.DMA,
        ],
    )
    def deg_kernel(dst_hbm, out_hbm, dst_v, rid_v, ones_v, robuf, acc_sp,
                   sem):
        c = lax.axis_index("c")
        s = lax.axis_index("s")
        wid = s * NC + c
        pltpu.sync_copy(dst_hbm.at[wid], dst_v)
        _fill_rid(rid_v, nm, s * rpt)
        _zero_rows(robuf, CHUNK, LANES)

        def zero(j, _):
            pltpu.sync_copy(robuf, acc_sp.at[rid_v.at[j]])
            return 0

        lax.fori_loop(0, nm, zero, 0)
        ones = jnp.ones((LANES,), jnp.float32)

        def f1(i, _):
            ones_v[i, :] = ones
            return 0

        lax.fori_loop(0, CHUNK, f1, 0)
        plsc.subcore_barrier()

        def chunk(i, _):
            pltpu.sync_copy(ones_v, acc_sp.at[dst_v.at[i]], add=True)
            return 0

        lax.fori_loop(0, c32, chunk, 0)
        plsc.subcore_barrier()

        def rd(j, _):
            pltpu.async_copy(acc_sp.at[rid_v.at[j]], robuf, sem).wait()
            pltpu.sync_copy(
                robuf, out_hbm.at[c, pl.ds(s * rpt + j * CHUNK, CHUNK)])
            return 0

        lax.fori_loop(0, nm, rd, 0)

    return deg_kernel


def _make_scatter(npad, c32, h):
    rpt = npad // NS
    nm = rpt // CHUNK
    mesh = plsc.VectorSubcoreMesh(core_axis_name="c", subcore_axis_name="s")

    @functools.partial(
        pl.kernel,
        mesh=mesh,
        out_type=jax.ShapeDtypeStruct((NC, npad, h), jnp.float32),
        compiler_params=pltpu.CompilerParams(use_tc_tiling_on_sc=False),
        scratch_types=[
            pltpu.VMEM((c32, CHUNK), jnp.int32),
            pltpu.VMEM((c32, CHUNK), jnp.int32),
            pltpu.VMEM((c32, CHUNK), jnp.int32),
            pltpu.VMEM((nm, CHUNK), jnp.int32),
            pltpu.VMEM((CHUNK, h), jnp.float32),
            pltpu.VMEM((CHUNK, h), jnp.float32),
            pltpu.VMEM_SHARED((npad, h), jnp.float32),
            pltpu.SemaphoreType.DMA,
        ],
    )
    def scatter_kernel(pk_hbm, y_hbm, out_hbm,
                       pk_v, src_v, dst_v, rid_v, rows_v, robuf, acc_sp,
                       sem):
        c = lax.axis_index("c")
        s = lax.axis_index("s")
        wid = s * NC + c
        pltpu.sync_copy(pk_hbm.at[wid], pk_v)
        _fill_rid(rid_v, nm, s * rpt)
        _zero_rows(robuf, CHUNK, h)
        kpc = CHUNK // LANES

        def unpk(t, _):
            i = t // kpc
            k = t % kpc
            v = pk_v[i, pl.ds(k * LANES, LANES)]
            src_v[i, pl.ds(k * LANES, LANES)] = lax.shift_right_logical(
                v, 14)
            dst_v[i, pl.ds(k * LANES, LANES)] = jnp.bitwise_and(v, 16383)
            return 0

        lax.fori_loop(0, c32 * kpc, unpk, 0)

        def zero(j, _):
            pltpu.sync_copy(robuf, acc_sp.at[rid_v.at[j]])
            return 0

        lax.fori_loop(0, nm, zero, 0)
        plsc.subcore_barrier()

        def chunk(i, _):
            pltpu.async_copy(y_hbm.at[src_v.at[i]], rows_v, sem).wait()
            pltpu.sync_copy(rows_v, acc_sp.at[dst_v.at[i]], add=True)
            return 0

        lax.fori_loop(0, c32, chunk, 0)
        plsc.subcore_barrier()

        def rd(j, _):
            pltpu.async_copy(acc_sp.at[rid_v.at[j]], robuf, sem).wait()
            pltpu.sync_copy(
                robuf, out_hbm.at[c, pl.ds(s * rpt + j * CHUNK, CHUNK)])
            return 0

        lax.fori_loop(0, nm, rd, 0)

    return scatter_kernel


def _y1_body(x_ref, w_ref, deg_ref, y_ref, dis_ref):
    deg = deg_ref[0, :, :1] + deg_ref[1, :, :1] + 1.0
    dis = lax.rsqrt(deg)
    dis_ref[...] = dis
    y_ref[...] = (
        jnp.dot(x_ref[...], w_ref[...], preferred_element_type=jnp.float32)
        * dis
    )


def _mid_body(acc_ref, y1_ref, dis_ref, b1_ref, w2_ref, o_ref):
    a = acc_ref[0] + acc_ref[1] + y1_ref[...]
    dis = dis_ref[...]
    hid = jnp.maximum(dis * a + b1_ref[...], 0.0)
    o_ref[...] = dis * jnp.dot(
        hid, w2_ref[...], preferred_element_type=jnp.float32
    )


def _out_body(acc_ref, y2_ref, dis_ref, b2_ref, o_ref):
    a = acc_ref[0] + acc_ref[1] + y2_ref[...]
    o_ref[...] = jnp.maximum(dis_ref[...] * a + b2_ref[...], 0.0)


def kernel(x, edge_index, W1, b1, W2, b2):
    n, d = x.shape
    e = edge_index.shape[1]
    h1 = W1.shape[1]
    h2 = W2.shape[1]
    hp = 16

    R = 2048
    npad = -(-(n + 1) // R) * R
    eq = NC * NS * CHUNK * 8
    epad = -(-e // eq) * eq

    src = jnp.concatenate(
        [edge_index[0], jnp.zeros((epad - e,), jnp.int32)])
    dst = jnp.concatenate(
        [edge_index[1], jnp.full((epad - e,), n, jnp.int32)])
    pk32 = (jnp.left_shift(src, 14) | dst).reshape(NC * NS, -1, CHUNK)
    dst32 = dst.reshape(NC * NS, -1, CHUNK)
    c32 = pk32.shape[1]

    xp = jnp.pad(x, ((0, npad - n), (0, 0)))
    w2p = jnp.pad(W2, ((0, 0), (0, hp - h2)))
    b1r = b1.reshape(1, h1)
    b2p = jnp.pad(b2, (0, hp - h2)).reshape(1, hp)

    deg = _make_deg(npad, c32)(dst32)
    grid = (npad // R,)

    y1, dis2 = pl.pallas_call(
        _y1_body,
        grid=grid,
        in_specs=[
            pl.BlockSpec((R, d), lambda b: (b, 0)),
            pl.BlockSpec((d, h1), lambda b: (0, 0)),
            pl.BlockSpec((NC, R, LANES), lambda b: (0, b, 0)),
        ],
        out_specs=[
            pl.BlockSpec((R, h1), lambda b: (b, 0)),
            pl.BlockSpec((R, 1), lambda b: (b, 0)),
        ],
        out_shape=[
            jax.ShapeDtypeStruct((npad, h1), jnp.float32),
            jax.ShapeDtypeStruct((npad, 1), jnp.float32),
        ],
    )(xp, W1, deg)

    acc1 = _make_scatter(npad, c32, h1)(pk32, y1)

    y2 = pl.pallas_call(
        _mid_body,
        grid=grid,
        in_specs=[
            pl.BlockSpec((NC, R, h1), lambda b: (0, b, 0)),
            pl.BlockSpec((R, h1), lambda b: (b, 0)),
            pl.BlockSpec((R, 1), lambda b: (b, 0)),
            pl.BlockSpec((1, h1), lambda b: (0, 0)),
            pl.BlockSpec((h1, hp), lambda b: (0, 0)),
        ],
        out_specs=pl.BlockSpec((R, hp), lambda b: (b, 0)),
        out_shape=jax.ShapeDtypeStruct((npad, hp), jnp.float32),
    )(acc1, y1, dis2, b1r, w2p)

    acc2 = _make_scatter(npad, c32, hp)(pk32, y2)

    out = pl.pallas_call(
        _out_body,
        grid=grid,
        in_specs=[
            pl.BlockSpec((NC, R, hp), lambda b: (0, b, 0)),
            pl.BlockSpec((R, hp), lambda b: (b, 0)),
            pl.BlockSpec((R, 1), lambda b: (b, 0)),
            pl.BlockSpec((1, hp), lambda b: (0, 0)),
        ],
        out_specs=pl.BlockSpec((R, hp), lambda b: (b, 0)),
        out_shape=jax.ShapeDtypeStruct((npad, hp), jnp.float32),
    )(acc2, y2, dis2, b2p)

    return out[:n, :h2]
